# baseline (speedup 1.0000x reference)
import jax
import jax.numpy as jnp
import numpy as np
from jax import lax
from jax.experimental import pallas as pl
from jax.experimental.pallas import tpu as pltpu
from jax.experimental.pallas import tpu_sc as plsc

_BETA = 0.25
_T = 4
_B = 32
_N = 384 * 16 * 16
_K = _N // 2
_L = 16
_CH = 16 * 384
_NCH = _N // _CH
_HI = np.uint32(0x80000000)
_ALL1 = np.uint32(0xFFFFFFFF)
_PASSES = ((20, 12), (10, 10), (0, 10))


def _sort_key(m):
    bu = plsc.bitcast(m, jnp.uint32)
    flip = jnp.where(bu >= _HI, _ALL1, _HI)
    return bu ^ flip


def _inv_key(key):
    flip = jnp.where(key >= _HI, _HI, _ALL1)
    return plsc.bitcast(key ^ flip, jnp.float32)


def _scan_bins(hist, k_rem, lanes, nbins):
    groups = nbins // _L

    def jbody(jj, carry):
        acc, found, vstar, above = carry
        j = groups - 1 - jj
        c_vec = hist[pl.ds(j * _L, _L)]
        tot = jnp.sum(c_vec)
        s_local = lax.rev(plsc.cumsum(lax.rev(c_vec, (0,))), (0,))
        hit = jnp.logical_and(found == 0, acc + tot >= k_rem)
        mvec = (acc + s_local) >= k_rem
        pc = plsc.all_reduce_population_count(mvec)
        i0 = jnp.max(pc) - 1
        cv_at = jnp.sum(jnp.where(lanes == i0, c_vec, 0))
        sv_at = jnp.sum(jnp.where(lanes == i0, s_local, 0))
        above_j = acc + sv_at - cv_at
        vstar = jnp.where(hit, j * _L + i0, vstar)
        above = jnp.where(hit, above_j, above)
        found = jnp.where(hit, 1, found)
        return (acc + tot, found, vstar, above)

    _, _, vstar, above = lax.fori_loop(
        0, groups, jbody,
        (jnp.int32(0), jnp.int32(0), jnp.int32(0), jnp.int32(0)))
    return vstar, above


def _lif_body(x_hbm, out_hbm, mem, hist, xbuf, sbuf, in_sem, out_sem):
    c = lax.axis_index("c")
    s = lax.axis_index("s")
    b = s * 2 + c
    lanes = lax.iota(jnp.int32, _L)
    ones_i32 = jnp.ones((_L,), jnp.int32)
    zeros_i = jnp.zeros((_L,), jnp.int32)

    def zhist(nbins):
        @plsc.parallel_loop(0, nbins // _L, unroll=8)
        def _(i):
            hist[pl.ds(i * _L, _L)] = zeros_i

    def start_in(t, ch):
        pltpu.async_copy(x_hbm.at[t, b, ch, :, :], xbuf.at[ch % 2], in_sem)

    def wait_in(t, ch):
        pltpu.make_async_copy(
            x_hbm.at[t, b, ch, :, :], xbuf.at[ch % 2], in_sem).wait()

    def start_out(t, ch):
        pltpu.async_copy(sbuf.at[ch % 2], out_hbm.at[t, b, ch, :, :],
                         out_sem)

    def wait_out(t, ch):
        pltpu.make_async_copy(
            sbuf.at[ch % 2], out_hbm.at[t, b, ch, :, :], out_sem).wait()

    sh0, w0 = _PASSES[0]
    thr_key = jnp.broadcast_to(jnp.uint32(0), (_L,))
    start_in(0, 0)
    for t in range(_T):
        zhist(1 << w0)

        def chunk_body(ch, _, t=t, thr_key=thr_key):
            wait_in(t, ch)

            @pl.when(ch + 1 < _NCH)
            def _():
                start_in(t, ch + 1)

            bi = ch % 2
            if t == 0:
                @plsc.parallel_loop(0, 16, unroll=2)
                def _(q, ch=ch, bi=bi):
                    for r in range(24):
                        gidx = pl.ds(ch * _CH + q * 384 + r * _L, _L)
                        m = xbuf[bi, q, pl.ds(r * _L, _L)] + jnp.float32(0.0)
                        key = _sort_key(m)
                        mem[gidx] = key
                        digit = (key >> jnp.uint32(sh0)).astype(jnp.int32)
                        plsc.addupdate_scatter(hist, [digit], ones_i32)
            else:
                @pl.when(ch >= 2)
                def _():
                    wait_out(t - 1, ch - 2)

                @plsc.parallel_loop(0, 16, unroll=2)
                def _(q, ch=ch, bi=bi, thr_key=thr_key):
                    for r in range(24):
                        gidx = pl.ds(ch * _CH + q * 384 + r * _L, _L)
                        key = mem[gidx]
                        ge = key >= thr_key
                        sbuf[bi, q, pl.ds(r * _L, _L)] = jnp.where(
                            ge, jnp.float32(1.0), jnp.float32(0.0))
                        mk = _inv_key(jnp.where(ge, _HI, key))
                        m = (jnp.float32(_BETA) * mk
                             + xbuf[bi, q, pl.ds(r * _L, _L)])
                        nkey = _sort_key(m)
                        mem[gidx] = nkey
                        digit = (nkey >> jnp.uint32(sh0)).astype(jnp.int32)
                        plsc.addupdate_scatter(hist, [digit], ones_i32)
                start_out(t - 1, ch)
            return 0

        lax.fori_loop(0, _NCH, chunk_body, 0)
        if t < _T - 1:
            start_in(t + 1, 0)
        if t > 0:
            wait_out(t - 1, _NCH - 2)
            wait_out(t - 1, _NCH - 1)

        k_rem = jnp.int32(_K)
        prefix = jnp.uint32(0)
        for pi, (shift, width) in enumerate(_PASSES):
            if pi > 0:
                zhist(1 << width)
                hi_sh = jnp.uint32(shift + width)
                pref_hi = prefix >> hi_sh
                dmask = jnp.uint32((1 << width) - 1)

                @plsc.parallel_loop(0, _N // _L, unroll=8)
                def _(i, shift=shift, hi_sh=hi_sh, pref_hi=pref_hi,
                      dmask=dmask):
                    key = mem[pl.ds(i * _L, _L)]
                    match = (key >> hi_sh) == pref_hi
                    digit = ((key >> jnp.uint32(shift)) & dmask
                             ).astype(jnp.int32)
                    plsc.addupdate_scatter(hist, [digit], ones_i32,
                                           mask=match)
            vstar, above = _scan_bins(hist, k_rem, lanes, 1 << width)
            k_rem = k_rem - above
            prefix = prefix | (vstar.astype(jnp.uint32) << jnp.uint32(shift))

        thr_key = jnp.broadcast_to(prefix, (_L,))

    def tail_body(ch, _, thr_key=thr_key):
        @pl.when(ch >= 2)
        def _():
            wait_out(_T - 1, ch - 2)

        bi = ch % 2

        @plsc.parallel_loop(0, 16, unroll=2)
        def _(q, ch=ch, bi=bi, thr_key=thr_key):
            for r in range(24):
                key = mem[pl.ds(ch * _CH + q * 384 + r * _L, _L)]
                sbuf[bi, q, pl.ds(r * _L, _L)] = jnp.where(
                    key >= thr_key, jnp.float32(1.0), jnp.float32(0.0))
        start_out(_T - 1, ch)
        return 0

    lax.fori_loop(0, _NCH, tail_body, 0)
    wait_out(_T - 1, _NCH - 2)
    wait_out(_T - 1, _NCH - 1)


def kernel(x):
    lif = pl.kernel(
        _lif_body,
        out_type=jax.ShapeDtypeStruct((_T, _B, 16, 16, 384), jnp.float32),
        mesh=plsc.VectorSubcoreMesh(core_axis_name="c", subcore_axis_name="s"),
        compiler_params=pltpu.CompilerParams(needs_layout_passes=False,
                                             use_tc_tiling_on_sc=True),
        scratch_types=[
            pltpu.VMEM((_N,), jnp.uint32),
            pltpu.VMEM((1 << _PASSES[0][1],), jnp.int32),
            pltpu.VMEM((2, 16, 384), jnp.float32),
            pltpu.VMEM((2, 16, 384), jnp.float32),
            pltpu.SemaphoreType.DMA,
            pltpu.SemaphoreType.DMA,
        ],
    )
    xt = lax.transpose(x, (0, 1, 3, 4, 2))
    y = lif(xt)
    return lax.transpose(y, (0, 1, 4, 2, 3))


# speedup vs baseline: 1.0716x; 1.0716x over previous
import jax
import jax.numpy as jnp
import numpy as np
from jax import lax
from jax.experimental import pallas as pl
from jax.experimental.pallas import tpu as pltpu
from jax.experimental.pallas import tpu_sc as plsc

_BETA = 0.25
_T = 4
_B = 32
_N = 384 * 16 * 16
_K = _N // 2
_L = 16
_CH = 16 * 384
_NCH = _N // _CH
_HI = np.uint32(0x80000000)
_ALL1 = np.uint32(0xFFFFFFFF)
_PASSES = ((20, 12), (10, 10), (0, 10))
_NBINS = 1 << _PASSES[0][1]
_CAND0 = 1 << _PASSES[1][1]
_CAP = _NBINS - _CAND0 - 16


def _sort_key(m):
    bu = plsc.bitcast(m, jnp.uint32)
    flip = jnp.where(bu >= _HI, _ALL1, _HI)
    return bu ^ flip


def _scan_bins(hist, k_rem, lanes, nbins):
    groups = nbins // _L

    def jbody(jj, carry):
        acc, found, vstar, above, csel = carry
        j = groups - 1 - jj
        c_vec = hist[pl.ds(j * _L, _L)]
        tot = jnp.sum(c_vec)
        s_local = lax.rev(plsc.cumsum(lax.rev(c_vec, (0,))), (0,))
        hit = jnp.logical_and(found == 0, acc + tot >= k_rem)
        mvec = (acc + s_local) >= k_rem
        pc = plsc.all_reduce_population_count(mvec)
        i0 = jnp.max(pc) - 1
        cv_at = jnp.sum(jnp.where(lanes == i0, c_vec, 0))
        sv_at = jnp.sum(jnp.where(lanes == i0, s_local, 0))
        above_j = acc + sv_at - cv_at
        vstar = jnp.where(hit, j * _L + i0, vstar)
        above = jnp.where(hit, above_j, above)
        csel = jnp.where(hit, cv_at, csel)
        found = jnp.where(hit, 1, found)
        return (acc + tot, found, vstar, above, csel)

    _, _, vstar, above, csel = lax.fori_loop(
        0, groups, jbody,
        (jnp.int32(0), jnp.int32(0), jnp.int32(0), jnp.int32(0),
         jnp.int32(0)))
    return vstar, above, csel


def _lif_body(x_hbm, out_hbm, mem, hist, xbuf, sbuf, in_sem, out_sem):
    c = lax.axis_index("c")
    s = lax.axis_index("s")
    b = s * 2 + c
    lanes = lax.iota(jnp.int32, _L)
    ones_i32 = jnp.ones((_L,), jnp.int32)
    zeros_i = jnp.zeros((_L,), jnp.int32)

    def zhist(nbins):
        @plsc.parallel_loop(0, nbins // _L, unroll=8)
        def _(i):
            hist[pl.ds(i * _L, _L)] = zeros_i

    def start_in(t, ch):
        pltpu.async_copy(x_hbm.at[t, b, ch, :, :], xbuf.at[ch % 2], in_sem)

    def wait_in(t, ch):
        pltpu.make_async_copy(
            x_hbm.at[t, b, ch, :, :], xbuf.at[ch % 2], in_sem).wait()

    def start_out(t, ch):
        pltpu.async_copy(sbuf.at[ch % 2], out_hbm.at[t, b, ch, :, :],
                         out_sem)

    def wait_out(t, ch):
        pltpu.make_async_copy(
            sbuf.at[ch % 2], out_hbm.at[t, b, ch, :, :], out_sem).wait()

    sh0, w0 = _PASSES[0]
    thr = jnp.broadcast_to(jnp.float32(0.0), (_L,))
    start_in(0, 0)
    for t in range(_T):
        zhist(_NBINS)

        def chunk_body(ch, _, t=t, thr=thr):
            wait_in(t, ch)

            @pl.when(ch + 1 < _NCH)
            def _():
                start_in(t, ch + 1)

            bi = ch % 2
            if t == 0:
                @plsc.parallel_loop(0, 16, unroll=2)
                def _(q, ch=ch, bi=bi):
                    for r in range(24):
                        gidx = pl.ds(ch * _CH + q * 384 + r * _L, _L)
                        m = xbuf[bi, q, pl.ds(r * _L, _L)]
                        mem[gidx] = m
                        key = _sort_key(m)
                        digit = (key >> jnp.uint32(sh0)).astype(jnp.int32)
                        plsc.addupdate_scatter(hist, [digit], ones_i32)
            else:
                @pl.when(ch >= 2)
                def _():
                    wait_out(t - 1, ch - 2)

                @plsc.parallel_loop(0, 16, unroll=2)
                def _(q, ch=ch, bi=bi, thr=thr):
                    for r in range(24):
                        gidx = pl.ds(ch * _CH + q * 384 + r * _L, _L)
                        m = mem[gidx]
                        ge = m >= thr
                        sbuf[bi, q, pl.ds(r * _L, _L)] = jnp.where(
                            ge, jnp.float32(1.0), jnp.float32(0.0))
                        m = (jnp.float32(_BETA)
                             * jnp.where(ge, jnp.float32(0.0), m)
                             + xbuf[bi, q, pl.ds(r * _L, _L)])
                        mem[gidx] = m
                        key = _sort_key(m)
                        digit = (key >> jnp.uint32(sh0)).astype(jnp.int32)
                        plsc.addupdate_scatter(hist, [digit], ones_i32)
                start_out(t - 1, ch)
            return 0

        lax.fori_loop(0, _NCH, chunk_body, 0)
        if t < _T - 1:
            start_in(t + 1, 0)
        if t > 0:
            wait_out(t - 1, _NCH - 2)
            wait_out(t - 1, _NCH - 1)

        k_rem = jnp.int32(_K)
        vstar, above, csel = _scan_bins(hist, k_rem, lanes, _NBINS)
        k_rem = k_rem - above
        prefix = vstar.astype(jnp.uint32) << jnp.uint32(_PASSES[0][0])
        fits = csel <= _CAP

        zhist(_CAND0)
        sh1 = jnp.uint32(_PASSES[1][0])
        hi1 = jnp.uint32(_PASSES[1][0] + _PASSES[1][1])
        pref1 = prefix >> hi1
        dm1 = jnp.uint32((1 << _PASSES[1][1]) - 1)

        @plsc.parallel_loop(0, _N // _L, unroll=1,
                            carry=jnp.int32(_CAND0))
        def hloop(i, off, fits=fits, pref1=pref1):
            key = _sort_key(mem[pl.ds(i * _L, _L)])
            match = (key >> hi1) == pref1
            digit = ((key >> sh1) & dm1).astype(jnp.int32)
            plsc.addupdate_scatter(hist, [digit], ones_i32, mask=match)
            ok = jnp.logical_and(match, fits)
            plsc.store_compressed(hist.at[pl.ds(off, _L)],
                                  plsc.bitcast(key, jnp.int32), mask=ok)
            pc = jnp.max(plsc.all_reduce_population_count(ok))
            return off + pc

        vstar, above, _ = _scan_bins(hist, k_rem, lanes, _CAND0)
        k_rem = k_rem - above
        prefix = prefix | (vstar.astype(jnp.uint32) << sh1)

        zhist(_CAND0)
        hi2 = jnp.uint32(_PASSES[2][1])
        pref2 = prefix >> hi2
        dm2 = jnp.uint32((1 << _PASSES[2][1]) - 1)

        def compact_pass(_):
            nv = (csel + (_L - 1)) // _L

            def cbody(i, _):
                key = plsc.bitcast(hist[pl.ds(_CAND0 + i * _L, _L)],
                                   jnp.uint32)
                valid = (i * _L + lanes) < csel
                match = jnp.logical_and((key >> hi2) == pref2, valid)
                digit = (key & dm2).astype(jnp.int32)
                plsc.addupdate_scatter(hist, [digit], ones_i32, mask=match)
                return 0

            lax.fori_loop(0, nv, cbody, 0)
            return 0

        def full_pass(_):
            @plsc.parallel_loop(0, _N // _L, unroll=8)
            def _(i, pref2=pref2):
                key = _sort_key(mem[pl.ds(i * _L, _L)])
                match = (key >> hi2) == pref2
                digit = (key & dm2).astype(jnp.int32)
                plsc.addupdate_scatter(hist, [digit], ones_i32, mask=match)
            return 0

        lax.cond(fits, compact_pass, full_pass, 0)

        vstar, above, _ = _scan_bins(hist, k_rem, lanes, _CAND0)
        prefix = prefix | vstar.astype(jnp.uint32)

        thr_bits = jnp.where(prefix >= _HI, prefix ^ _HI, prefix ^ _ALL1)
        thr = plsc.bitcast(jnp.broadcast_to(thr_bits, (_L,)), jnp.float32)

    def tail_body(ch, _, thr=thr):
        @pl.when(ch >= 2)
        def _():
            wait_out(_T - 1, ch - 2)

        bi = ch % 2

        @plsc.parallel_loop(0, 16, unroll=2)
        def _(q, ch=ch, bi=bi, thr=thr):
            for r in range(24):
                m = mem[pl.ds(ch * _CH + q * 384 + r * _L, _L)]
                sbuf[bi, q, pl.ds(r * _L, _L)] = jnp.where(
                    m >= thr, jnp.float32(1.0), jnp.float32(0.0))
        start_out(_T - 1, ch)
        return 0

    lax.fori_loop(0, _NCH, tail_body, 0)
    wait_out(_T - 1, _NCH - 2)
    wait_out(_T - 1, _NCH - 1)


def kernel(x):
    lif = pl.kernel(
        _lif_body,
        out_type=jax.ShapeDtypeStruct((_T, _B, 16, 16, 384), jnp.float32),
        mesh=plsc.VectorSubcoreMesh(core_axis_name="c", subcore_axis_name="s"),
        compiler_params=pltpu.CompilerParams(needs_layout_passes=False,
                                             use_tc_tiling_on_sc=True),
        scratch_types=[
            pltpu.VMEM((_N,), jnp.float32),
            pltpu.VMEM((_NBINS,), jnp.int32),
            pltpu.VMEM((2, 16, 384), jnp.float32),
            pltpu.VMEM((2, 16, 384), jnp.float32),
            pltpu.SemaphoreType.DMA,
            pltpu.SemaphoreType.DMA,
        ],
    )
    xt = lax.transpose(x, (0, 1, 3, 4, 2))
    y = lif(xt)
    return lax.transpose(y, (0, 1, 4, 2, 3))


# speedup vs baseline: 1.2501x; 1.1666x over previous
import jax
import jax.numpy as jnp
import numpy as np
from jax import lax
from jax.experimental import pallas as pl
from jax.experimental.pallas import tpu as pltpu
from jax.experimental.pallas import tpu_sc as plsc

_BETA = 0.25
_T = 4
_B = 32
_N = 384 * 16 * 16
_K = _N // 2
_L = 16
_CH = 16 * 384
_NCH = _N // _CH
_HI = np.uint32(0x80000000)
_ALL1 = np.uint32(0xFFFFFFFF)
_PASSES = ((20, 12), (10, 10), (0, 10))


def _sort_key(m):
    bu = plsc.bitcast(m, jnp.uint32)
    flip = jnp.where(bu >= _HI, _ALL1, _HI)
    return bu ^ flip


def _scan_bins(hist, k_rem, lanes, nbins):
    groups = nbins // _L

    def jbody(jj, carry):
        acc, found, vstar, above = carry
        j = groups - 1 - jj
        c_vec = hist[pl.ds(j * _L, _L)]
        tot = jnp.sum(c_vec)
        s_local = lax.rev(plsc.cumsum(lax.rev(c_vec, (0,))), (0,))
        hit = jnp.logical_and(found == 0, acc + tot >= k_rem)
        mvec = (acc + s_local) >= k_rem
        pc = plsc.all_reduce_population_count(mvec)
        i0 = jnp.max(pc) - 1
        cv_at = jnp.sum(jnp.where(lanes == i0, c_vec, 0))
        sv_at = jnp.sum(jnp.where(lanes == i0, s_local, 0))
        above_j = acc + sv_at - cv_at
        vstar = jnp.where(hit, j * _L + i0, vstar)
        above = jnp.where(hit, above_j, above)
        found = jnp.where(hit, 1, found)
        return (acc + tot, found, vstar, above)

    _, _, vstar, above = lax.fori_loop(
        0, groups, jbody,
        (jnp.int32(0), jnp.int32(0), jnp.int32(0), jnp.int32(0)))
    return vstar, above


def _lif_body(x_hbm, out_hbm, mem, hist, xbuf, sbuf, in_sem, out_sem):
    c = lax.axis_index("c")
    s = lax.axis_index("s")
    b = s * 2 + c
    lanes = lax.iota(jnp.int32, _L)
    ones_i32 = jnp.ones((_L,), jnp.int32)
    zeros_i = jnp.zeros((_L,), jnp.int32)

    def zhist(nbins):
        @plsc.parallel_loop(0, nbins // _L, unroll=8)
        def _(i):
            hist[pl.ds(i * _L, _L)] = zeros_i

    def start_in(t, ch):
        pltpu.async_copy(x_hbm.at[t, b, ch, :, :], xbuf.at[ch % 2], in_sem)

    def wait_in(t, ch):
        pltpu.make_async_copy(
            x_hbm.at[t, b, ch, :, :], xbuf.at[ch % 2], in_sem).wait()

    def start_out(t, ch):
        pltpu.async_copy(sbuf.at[ch % 2], out_hbm.at[t, b, ch, :, :],
                         out_sem)

    def wait_out(t, ch):
        pltpu.make_async_copy(
            sbuf.at[ch % 2], out_hbm.at[t, b, ch, :, :], out_sem).wait()

    sh0, w0 = _PASSES[0]
    thr = jnp.broadcast_to(jnp.float32(0.0), (_L,))
    start_in(0, 0)
    for t in range(_T):
        zhist(1 << w0)

        def chunk_body(ch, _, t=t, thr=thr):
            wait_in(t, ch)

            @pl.when(ch + 1 < _NCH)
            def _():
                start_in(t, ch + 1)

            bi = ch % 2
            if t == 0:
                @plsc.parallel_loop(0, 16, unroll=2)
                def _(q, ch=ch, bi=bi):
                    for r in range(24):
                        gidx = pl.ds(ch * _CH + q * 384 + r * _L, _L)
                        m = xbuf[bi, q, pl.ds(r * _L, _L)]
                        mem[gidx] = m
                        key = _sort_key(m)
                        digit = (key >> jnp.uint32(sh0)).astype(jnp.int32)
                        plsc.addupdate_scatter(hist, [digit], ones_i32)
            else:
                @pl.when(ch >= 2)
                def _():
                    wait_out(t - 1, ch - 2)

                @plsc.parallel_loop(0, 16, unroll=2)
                def _(q, ch=ch, bi=bi, thr=thr):
                    for r in range(24):
                        gidx = pl.ds(ch * _CH + q * 384 + r * _L, _L)
                        m = mem[gidx]
                        ge = m >= thr
                        sbuf[bi, q, pl.ds(r * _L, _L)] = jnp.where(
                            ge, jnp.float32(1.0), jnp.float32(0.0))
                        m = (jnp.float32(_BETA)
                             * jnp.where(ge, jnp.float32(0.0), m)
                             + xbuf[bi, q, pl.ds(r * _L, _L)])
                        mem[gidx] = m
                        key = _sort_key(m)
                        digit = (key >> jnp.uint32(sh0)).astype(jnp.int32)
                        plsc.addupdate_scatter(hist, [digit], ones_i32)
                start_out(t - 1, ch)
            return 0

        lax.fori_loop(0, _NCH, chunk_body, 0)
        if t < _T - 1:
            start_in(t + 1, 0)
        if t > 0:
            wait_out(t - 1, _NCH - 2)
            wait_out(t - 1, _NCH - 1)

        k_rem = jnp.int32(_K)
        prefix = jnp.uint32(0)
        for pi, (shift, width) in enumerate(_PASSES):
            if pi > 0:
                zhist(1 << width)
                hi_sh = jnp.uint32(shift + width)
                pref_hi = prefix >> hi_sh
                dmask = jnp.uint32((1 << width) - 1)

                @plsc.parallel_loop(0, _N // _L, unroll=8)
                def _(i, shift=shift, hi_sh=hi_sh, pref_hi=pref_hi,
                      dmask=dmask):
                    key = _sort_key(mem[pl.ds(i * _L, _L)])
                    match = (key >> hi_sh) == pref_hi
                    digit = ((key >> jnp.uint32(shift)) & dmask
                             ).astype(jnp.int32)
                    plsc.addupdate_scatter(hist, [digit], ones_i32,
                                           mask=match)
            vstar, above = _scan_bins(hist, k_rem, lanes, 1 << width)
            k_rem = k_rem - above
            prefix = prefix | (vstar.astype(jnp.uint32) << jnp.uint32(shift))

        thr_bits = jnp.where(prefix >= _HI, prefix ^ _HI, prefix ^ _ALL1)
        thr = plsc.bitcast(jnp.broadcast_to(thr_bits, (_L,)), jnp.float32)

    def tail_body(ch, _, thr=thr):
        @pl.when(ch >= 2)
        def _():
            wait_out(_T - 1, ch - 2)

        bi = ch % 2

        @plsc.parallel_loop(0, 16, unroll=2)
        def _(q, ch=ch, bi=bi, thr=thr):
            for r in range(24):
                m = mem[pl.ds(ch * _CH + q * 384 + r * _L, _L)]
                sbuf[bi, q, pl.ds(r * _L, _L)] = jnp.where(
                    m >= thr, jnp.float32(1.0), jnp.float32(0.0))
        start_out(_T - 1, ch)
        return 0

    lax.fori_loop(0, _NCH, tail_body, 0)
    wait_out(_T - 1, _NCH - 2)
    wait_out(_T - 1, _NCH - 1)


def kernel(x):
    lif = pl.kernel(
        _lif_body,
        out_type=jax.ShapeDtypeStruct((_T, _B, 16, 16, 384), jnp.float32),
        mesh=plsc.VectorSubcoreMesh(core_axis_name="c", subcore_axis_name="s"),
        compiler_params=pltpu.CompilerParams(needs_layout_passes=False,
                                             use_tc_tiling_on_sc=True),
        scratch_types=[
            pltpu.VMEM((_N,), jnp.float32),
            pltpu.VMEM((1 << _PASSES[0][1],), jnp.int32),
            pltpu.VMEM((2, 16, 384), jnp.float32),
            pltpu.VMEM((2, 16, 384), jnp.float32),
            pltpu.SemaphoreType.DMA,
            pltpu.SemaphoreType.DMA,
        ],
    )
    xt = lax.transpose(x, (0, 1, 3, 4, 2))
    y = lif(xt)
    return lax.transpose(y, (0, 1, 4, 2, 3))
